# Initial kernel scaffold; baseline (speedup 1.0000x reference)
import jax
import jax.numpy as jnp
from jax.experimental import pallas as pl


def kernel(embeddings, target, triplets):
    raise NotImplementedError("write your pallas kernel here")


# speedup vs baseline: 7.2152x; 7.2152x over previous
import functools

import jax
import jax.numpy as jnp
from jax import lax
from jax.experimental import pallas as pl
from jax.experimental.pallas import tpu as pltpu
from jax.experimental.pallas import tpu_sc as plsc

_NC, _NS = 2, 16
_NW = _NC * _NS
_B = 32768
_D = 128
_TPW = _B // _NW
_C = 128
_NCHUNK = _TPW // _C
_EPS = 1e-12
_MARGIN = 0.2


def _sc_distances(embeddings, tri_r):
    mesh = plsc.VectorSubcoreMesh(core_axis_name="c", subcore_axis_name="s")
    out_type = (
        jax.ShapeDtypeStruct((_B,), jnp.float32),
        jax.ShapeDtypeStruct((_B,), jnp.float32),
    )
    scratch = [
        pltpu.VMEM((3, _NCHUNK, _C), jnp.int32),
        pltpu.VMEM((_C, _D), jnp.float32),
        pltpu.VMEM((_C, _D), jnp.float32),
        pltpu.VMEM((_C, _D), jnp.float32),
        pltpu.VMEM((_C, _D), jnp.float32),
        pltpu.VMEM((_C, _D), jnp.float32),
        pltpu.VMEM((_C, _D), jnp.float32),
        pltpu.VMEM((_TPW,), jnp.float32),
        pltpu.VMEM((_TPW,), jnp.float32),
        pltpu.SemaphoreType.DMA,
        pltpu.SemaphoreType.DMA,
    ]

    @functools.partial(pl.kernel, out_type=out_type, mesh=mesh,
                       scratch_types=scratch,
                       compiler_params=pltpu.CompilerParams(
                           needs_layout_passes=False))
    def k(emb_hbm, tri_hbm, ap2_hbm, an2_hbm,
          idx_v, a0, p0, n0, a1, p1, n1, ap2_v, an2_v, sem0, sem1):
        wid = lax.axis_index("s") * _NC + lax.axis_index("c")
        base = wid * _TPW
        bufs = ((a0, p0, n0), (a1, p1, n1))
        sems = (sem0, sem1)

        for comp in range(3):
            pltpu.sync_copy(tri_hbm.at[comp, wid], idx_v.at[comp])

        def fire(c, bs):
            return [
                pltpu.async_copy(emb_hbm.at[idx_v.at[comp, c]],
                                 bufs[bs][comp], sems[bs])
                for comp in range(3)
            ]

        def compute(c, bs):
            ba, bp, bn = bufs[bs]

            def group_body(g, _):
                def t_body(i, res):
                    res_ap, res_an = res
                    t = g * 16 + i
                    aap = jnp.zeros((16,), jnp.float32)
                    aan = jnp.zeros((16,), jnp.float32)
                    for dd in range(_D // 16):
                        sl = pl.ds(dd * 16, 16)
                        va = ba[t, sl]
                        vp = bp[t, sl]
                        vn = bn[t, sl]
                        vae = va + _EPS
                        dap = vae - vp
                        dan = vae - vn
                        aap = aap + dap * dap
                        aan = aan + dan * dan
                    lane = lax.iota(jnp.int32, 16) == i
                    res_ap = jnp.where(lane, jnp.sum(aap), res_ap)
                    res_an = jnp.where(lane, jnp.sum(aan), res_an)
                    return (res_ap, res_an)

                z = jnp.zeros((16,), jnp.float32)
                res_ap, res_an = lax.fori_loop(0, 16, t_body, (z, z))
                off = c * _C + g * 16
                ap2_v[pl.ds(off, 16)] = res_ap
                an2_v[pl.ds(off, 16)] = res_an
                return 0

            lax.fori_loop(0, _C // 16, group_body, 0)

        cps = fire(0, 0)
        for c in range(_NCHUNK):
            nxt = fire(c + 1, (c + 1) % 2) if c + 1 < _NCHUNK else None
            for cp in cps:
                cp.wait()
            compute(c, c % 2)
            cps = nxt

        pltpu.sync_copy(ap2_v, ap2_hbm.at[pl.ds(base, _TPW)])
        pltpu.sync_copy(an2_v, an2_hbm.at[pl.ds(base, _TPW)])

    return k(embeddings, tri_r)


def _tc_finish(ap_sq, an_sq):
    rows = _B // _D

    def body(ap2_ref, an2_ref, ap_ref, an_ref, loss_ref):
        ap = jnp.sqrt(ap2_ref[...])
        an = jnp.sqrt(an2_ref[...])
        ap_ref[...] = ap
        an_ref[...] = an
        loss_ref[0, 0] = jnp.sum(jnp.maximum(ap - an + _MARGIN, 0.0)) / _B

    ap_m, an_m, loss = pl.pallas_call(
        body,
        out_shape=(
            jax.ShapeDtypeStruct((rows, _D), jnp.float32),
            jax.ShapeDtypeStruct((rows, _D), jnp.float32),
            jax.ShapeDtypeStruct((1, 1), jnp.float32),
        ),
        out_specs=(
            pl.BlockSpec(memory_space=pltpu.VMEM),
            pl.BlockSpec(memory_space=pltpu.VMEM),
            pl.BlockSpec(memory_space=pltpu.SMEM),
        ),
    )(ap_sq.reshape(rows, _D), an_sq.reshape(rows, _D))
    return ap_m.reshape(_B), an_m.reshape(_B), loss[0, 0]


def kernel(embeddings, target, triplets):
    tri = triplets.astype(jnp.int32)
    tri_r = tri.T.reshape(3, _NW, _NCHUNK, _C)
    ap_sq, an_sq = _sc_distances(embeddings, tri_r)
    ap_d, an_d, loss = _tc_finish(ap_sq, an_sq)
    tdist = jnp.concatenate([ap_d, an_d], axis=0)
    ttarg = jnp.concatenate(
        [jnp.ones((_B,), jnp.float32), jnp.zeros((_B,), jnp.float32)], axis=0)
    return (loss, ap_d, an_d, tdist, ttarg)
